# baseline (speedup 1.0000x reference)
import functools

import jax
import jax.numpy as jnp
from jax import lax
from jax.experimental import pallas as pl
from jax.experimental.pallas import tpu as pltpu
from jax.experimental.pallas import tpu_sc as plsc

L = 16
NW = 32
BLK = 128
GROUPS = BLK // L
F = 2

_DUP_DN = lax.GatherDimensionNumbers(
    offset_dims=(), collapsed_slice_dims=(0,), start_index_map=(0,))


def _vgather(v, idx):
    return lax.gather(v, idx[:, None], _DUP_DN, (1,),
                      mode=lax.GatherScatterMode.PROMISE_IN_BOUNDS)


def _build_sc_call(npad, nblocks, res):
    per_w = nblocks * BLK
    zstride = res[2]
    ystride = res[1] * res[2]
    scale = tuple(float(r - 1) for r in res)
    hi = tuple(r - 2 for r in res)
    corner_offs = tuple(
        a * ystride + b * zstride + c
        for a in (0, 1) for b in (0, 1) for c in (0, 1)
    )
    mesh = plsc.VectorSubcoreMesh(core_axis_name="c", subcore_axis_name="s")

    @functools.partial(
        pl.kernel,
        mesh=mesh,
        out_type=jax.ShapeDtypeStruct((npad * F,), jnp.float32),
        scratch_types=[
            pltpu.VMEM((3, BLK), jnp.float32),
            pltpu.VMEM((3, BLK), jnp.float32),
            pltpu.VMEM((8 * F, BLK), jnp.int32),
            pltpu.VMEM((8 * F, BLK), jnp.float32),
            pltpu.VMEM((BLK * F,), jnp.float32),
            pltpu.SemaphoreType.DMA,
            pltpu.SemaphoreType.DMA,
        ],
    )
    def body(cx_h, cy_h, cz_h, tab_h, out_h, cbuf, wbuf, ibuf, gbuf, obuf,
             sem_in, sem_g):
        wid = lax.axis_index("s") * 2 + lax.axis_index("c")
        base = wid * per_w

        def block(b, carry):
            off = base + b * BLK
            cps = [
                pltpu.async_copy(cx_h.at[pl.ds(off, BLK)], cbuf.at[0], sem_in),
                pltpu.async_copy(cy_h.at[pl.ds(off, BLK)], cbuf.at[1], sem_in),
                pltpu.async_copy(cz_h.at[pl.ds(off, BLK)], cbuf.at[2], sem_in),
            ]
            for cp in cps:
                cp.wait()

            for g in range(GROUPS):
                s = pl.ds(g * L, L)
                fx = cbuf[0, s] * scale[0]
                fy = cbuf[1, s] * scale[1]
                fz = cbuf[2, s] * scale[2]
                ix = jnp.clip(fx.astype(jnp.int32), 0, hi[0])
                iy = jnp.clip(fy.astype(jnp.int32), 0, hi[1])
                iz = jnp.clip(fz.astype(jnp.int32), 0, hi[2])
                wbuf[0, s] = fx - ix.astype(jnp.float32)
                wbuf[1, s] = fy - iy.astype(jnp.float32)
                wbuf[2, s] = fz - iz.astype(jnp.float32)
                r2 = (ix * ystride + iy * zstride + iz) * F
                for c in range(8):
                    for f in range(F):
                        ibuf[c * F + f, s] = r2 + (corner_offs[c] * F + f)

            gcs = [
                pltpu.async_copy(tab_h.at[ibuf.at[j]], gbuf.at[j], sem_g)
                for j in range(8 * F)
            ]
            for gc in gcs:
                gc.wait()

            dup = lax.iota(jnp.int32, L) >> 1
            parity = (lax.iota(jnp.int32, L) & 1) == 1
            for g in range(GROUPS):
                s = pl.ds(g * L, L)
                wx = wbuf[0, s]
                wy = wbuf[1, s]
                wz = wbuf[2, s]
                ux = (1.0 - wx, wx)
                uy = (1.0 - wy, wy)
                uz = (1.0 - wz, wz)
                uxy = (ux[0] * uy[0], ux[0] * uy[1], ux[1] * uy[0], ux[1] * uy[1])
                w8 = tuple(uxy[c >> 1] * uz[c & 1] for c in range(8))
                acc0 = acc1 = None
                for c in range(8):
                    t0 = w8[c] * gbuf[c * F, s]
                    t1 = w8[c] * gbuf[c * F + 1, s]
                    acc0 = t0 if acc0 is None else acc0 + t0
                    acc1 = t1 if acc1 is None else acc1 + t1
                lo = jnp.where(parity, _vgather(acc1, dup), _vgather(acc0, dup))
                hi8 = dup + 8
                hv = jnp.where(parity, _vgather(acc1, hi8), _vgather(acc0, hi8))
                obuf[pl.ds(g * 2 * L, L)] = lo
                obuf[pl.ds(g * 2 * L + L, L)] = hv

            pltpu.sync_copy(obuf, out_h.at[pl.ds(off * F, BLK * F)])
            return carry

        lax.fori_loop(0, nblocks, block, 0)

    return body


def kernel(coords, table):
    n = coords.shape[1]
    res = table.shape[:-1]
    chunk = NW * BLK
    nblocks = -(-n // chunk)
    npad = nblocks * chunk
    pad = npad - n
    cx = jnp.pad(coords[0], (0, pad))
    cy = jnp.pad(coords[1], (0, pad))
    cz = jnp.pad(coords[2], (0, pad))
    tab1 = table.reshape(-1)
    out = _build_sc_call(npad, nblocks, res)(cx, cy, cz, tab1)
    return out.reshape(npad, F)[:n]
